# baseline (speedup 1.0000x reference)
import functools

import jax
import jax.numpy as jnp
from jax import lax
from jax.experimental import pallas as pl
from jax.experimental.pallas import tpu as pltpu
from jax.experimental.pallas import tpu_sc as plsc

N = 10000
E = 320000
D_IN = 128
D_H = 128
D_OUT = 64

NC = 2
NS = 16
NW = NC * NS

CHUNK = 100
R_PER_TILE = 100
EROWS = R_PER_TILE * NW

ROWS_PER_TILE = 625
N_ACC = NS * ROWS_PER_TILE

_MESH = plsc.VectorSubcoreMesh(
    core_axis_name="c", subcore_axis_name="s", num_cores=NC, num_subcores=NS)


@functools.partial(
    pl.kernel,
    out_type=(jax.ShapeDtypeStruct((NC, N_ACC, 16), jnp.float32),
              jax.ShapeDtypeStruct((NC, N_ACC, 16), jnp.float32)),
    mesh=_MESH,
    compiler_params=pltpu.CompilerParams(use_tc_tiling_on_sc=False),
    scratch_types=[
        pltpu.VMEM((R_PER_TILE, CHUNK), jnp.int32),
        pltpu.VMEM((R_PER_TILE, CHUNK), jnp.int32),
        pltpu.VMEM((CHUNK, 16), jnp.float32),
        pltpu.VMEM_SHARED((N_ACC, 16), jnp.float32),
        pltpu.VMEM_SHARED((N_ACC, 16), jnp.float32),
        [pltpu.SemaphoreType.DMA] * 4,
        [pltpu.SemaphoreType.DMA] * 4,
    ],
)
def _deg_kernel(src2_hbm, dst2_hbm, ones_hbm, zeros_hbm, dout_hbm, din_hbm,
                sidx, didx, ones_v, acc_out, acc_in, osems, isems):
  cid = lax.axis_index("c")
  sid = lax.axis_index("s")
  wid = cid * NS + sid
  base = sid * ROWS_PER_TILE
  rbase = wid * R_PER_TILE

  def start_scatters(i, s):
    pltpu.async_copy(ones_v, acc_out.at[sidx.at[i]], osems[s], add=True)
    pltpu.async_copy(ones_v, acc_in.at[didx.at[i]], isems[s], add=True)

  def wait_scatters(s):
    pltpu.make_async_copy(ones_v, acc_out.at[sidx.at[0]], osems[s]).wait()
    pltpu.make_async_copy(ones_v, acc_in.at[didx.at[0]], isems[s]).wait()

  pltpu.sync_copy(src2_hbm.at[pl.ds(rbase, R_PER_TILE)], sidx)
  pltpu.sync_copy(dst2_hbm.at[pl.ds(rbase, R_PER_TILE)], didx)
  pltpu.sync_copy(ones_hbm, ones_v)
  pltpu.sync_copy(zeros_hbm, acc_out.at[pl.ds(base, ROWS_PER_TILE)])
  pltpu.sync_copy(zeros_hbm, acc_in.at[pl.ds(base, ROWS_PER_TILE)])
  plsc.subcore_barrier()

  def loop_body(g, _):
    for k in range(4):
      i = 4 * g + k

      @pl.when(i >= 4)
      def _():
        wait_scatters(k)

      start_scatters(i, k)
    return 0

  lax.fori_loop(0, R_PER_TILE // 4, loop_body, 0)

  for k in range(4):
    wait_scatters(k)

  plsc.subcore_barrier()
  pltpu.sync_copy(acc_out.at[pl.ds(base, ROWS_PER_TILE)],
                  dout_hbm.at[cid, pl.ds(base, ROWS_PER_TILE)])
  pltpu.sync_copy(acc_in.at[pl.ds(base, ROWS_PER_TILE)],
                  din_hbm.at[cid, pl.ds(base, ROWS_PER_TILE)])


def _make_scatter_kernel(D):
  params = pltpu.CompilerParams(use_tc_tiling_on_sc=False)

  @functools.partial(
      pl.kernel,
      out_type=jax.ShapeDtypeStruct((NC, N_ACC, D), jnp.float32),
      mesh=_MESH,
      compiler_params=params,
      scratch_types=[
          pltpu.VMEM((R_PER_TILE, CHUNK), jnp.int32),
          pltpu.VMEM((R_PER_TILE, CHUNK), jnp.int32),
          pltpu.VMEM((CHUNK, D), jnp.float32),
          pltpu.VMEM((CHUNK, D), jnp.float32),
          pltpu.VMEM_SHARED((N_ACC, D), jnp.float32),
          [pltpu.SemaphoreType.DMA] * 2,
          [pltpu.SemaphoreType.DMA] * 2,
      ],
  )
  def _scatter_kernel(tab_hbm, src2_hbm, dst2_hbm, zeros_hbm, out_hbm,
                      sidx, didx, rows0, rows1, acc, gsems, ssems):
    cid = lax.axis_index("c")
    sid = lax.axis_index("s")
    wid = cid * NS + sid
    base = sid * ROWS_PER_TILE
    rbase = wid * R_PER_TILE

    rows = (rows0, rows1)

    def start_gather(i, s):
      pltpu.async_copy(tab_hbm.at[sidx.at[i]], rows[s], gsems[s])

    def wait_gather(s):
      pltpu.make_async_copy(tab_hbm.at[sidx.at[0]], rows[s], gsems[s]).wait()

    def start_scatter(i, s):
      pltpu.async_copy(rows[s], acc.at[didx.at[i]], ssems[s], add=True)

    def wait_scatter(s):
      pltpu.make_async_copy(rows[s], acc.at[didx.at[0]], ssems[s]).wait()

    pltpu.sync_copy(src2_hbm.at[pl.ds(rbase, R_PER_TILE)], sidx)
    start_gather(0, 0)
    pltpu.sync_copy(dst2_hbm.at[pl.ds(rbase, R_PER_TILE)], didx)
    pltpu.sync_copy(zeros_hbm, acc.at[pl.ds(base, ROWS_PER_TILE)])
    plsc.subcore_barrier()

    def loop_body(g, _):
      for k in range(2):
        i = 2 * g + k
        s = k
        o = 1 - k

        wait_gather(s)
        start_scatter(i, s)

        @pl.when(i + 1 < R_PER_TILE)
        def _():
          @pl.when(i >= 1)
          def _():
            wait_scatter(o)

          start_gather(i + 1, o)
      return 0

    lax.fori_loop(0, R_PER_TILE // 2, loop_body, 0)

    wait_scatter(0)
    wait_scatter(1)

    plsc.subcore_barrier()
    pltpu.sync_copy(acc.at[pl.ds(base, ROWS_PER_TILE)],
                    out_hbm.at[cid, pl.ds(base, ROWS_PER_TILE)])

  return _scatter_kernel


_scatter_128 = _make_scatter_kernel(D_H)
_scatter_64 = _make_scatter_kernel(D_OUT)


_BLK = 400
_GRID = N // _BLK


def _tc1_body(x_ref, w_ref, dout_ref, din_ref, xw_ref, rsin_ref, rsout_ref):
  rs_out = lax.rsqrt(jnp.maximum(dout_ref[0] + dout_ref[1], 1.0))
  rs_in = lax.rsqrt(jnp.maximum(din_ref[0] + din_ref[1], 1.0))
  rsout_ref[...] = rs_out
  rsin_ref[...] = rs_in
  xw = jnp.dot(x_ref[...], w_ref[...], preferred_element_type=jnp.float32)
  xw_ref[...] = xw * rs_out[:, 0:1]


def _tc1(x, w1, dout, din):
  return pl.pallas_call(
      _tc1_body,
      grid=(_GRID,),
      in_specs=[
          pl.BlockSpec((_BLK, D_IN), lambda i: (i, 0)),
          pl.BlockSpec((D_IN, D_H), lambda i: (0, 0)),
          pl.BlockSpec((NC, _BLK, 16), lambda i: (0, i, 0)),
          pl.BlockSpec((NC, _BLK, 16), lambda i: (0, i, 0)),
      ],
      out_specs=[
          pl.BlockSpec((_BLK, D_H), lambda i: (i, 0)),
          pl.BlockSpec((_BLK, 16), lambda i: (i, 0)),
          pl.BlockSpec((_BLK, 16), lambda i: (i, 0)),
      ],
      out_shape=[
          jax.ShapeDtypeStruct((N, D_H), jnp.float32),
          jax.ShapeDtypeStruct((N, 16), jnp.float32),
          jax.ShapeDtypeStruct((N, 16), jnp.float32),
      ],
  )(x, w1, dout, din)


def _tc2_body(p_ref, rsin_ref, b1_ref, w2_ref, rsout_ref, xw2_ref):
  agg = (p_ref[0] + p_ref[1]) * rsin_ref[...][:, 0:1]
  h = jnp.maximum(agg + b1_ref[...], 0.0)
  xw2 = jnp.dot(h, w2_ref[...], preferred_element_type=jnp.float32)
  xw2_ref[...] = xw2 * rsout_ref[...][:, 0:1]


def _tc2(parts1, rs_in, b1, w2, rs_out):
  return pl.pallas_call(
      _tc2_body,
      grid=(_GRID,),
      in_specs=[
          pl.BlockSpec((NC, _BLK, D_H), lambda i: (0, i, 0)),
          pl.BlockSpec((_BLK, 16), lambda i: (i, 0)),
          pl.BlockSpec((1, D_H), lambda i: (0, 0)),
          pl.BlockSpec((D_H, D_OUT), lambda i: (0, 0)),
          pl.BlockSpec((_BLK, 16), lambda i: (i, 0)),
      ],
      out_specs=pl.BlockSpec((_BLK, D_OUT), lambda i: (i, 0)),
      out_shape=jax.ShapeDtypeStruct((N, D_OUT), jnp.float32),
  )(parts1, rs_in, b1, w2, rs_out)


def _tc3_body(p_ref, rsin_ref, b2_ref, out_ref):
  agg = (p_ref[0] + p_ref[1]) * rsin_ref[...][:, 0:1]
  out_ref[...] = agg + b2_ref[...]


def _tc3(parts2, rs_in, b2):
  return pl.pallas_call(
      _tc3_body,
      grid=(_GRID,),
      in_specs=[
          pl.BlockSpec((NC, _BLK, D_OUT), lambda i: (0, i, 0)),
          pl.BlockSpec((_BLK, 16), lambda i: (i, 0)),
          pl.BlockSpec((1, D_OUT), lambda i: (0, 0)),
      ],
      out_specs=pl.BlockSpec((_BLK, D_OUT), lambda i: (i, 0)),
      out_shape=jax.ShapeDtypeStruct((N, D_OUT), jnp.float32),
  )(parts2, rs_in, b2)


def kernel(inputs, edge_index, W1, b1, W2, b2):
  src2 = edge_index[0].reshape(EROWS, CHUNK)
  dst2 = edge_index[1].reshape(EROWS, CHUNK)
  ones16 = jnp.ones((CHUNK, 16), jnp.float32)
  zeros16 = jnp.zeros((ROWS_PER_TILE, 16), jnp.float32)
  zeros128 = jnp.zeros((ROWS_PER_TILE, D_H), jnp.float32)
  zeros64 = jnp.zeros((ROWS_PER_TILE, D_OUT), jnp.float32)

  dout, din = _deg_kernel(src2, dst2, ones16, zeros16)
  xw1s, rs_in, rs_out = _tc1(inputs, W1, dout, din)
  parts1 = _scatter_128(xw1s, src2, dst2, zeros128)
  xw2s = _tc2(parts1, rs_in, b1.reshape(1, D_H), W2, rs_out)
  parts2 = _scatter_64(xw2s, src2, dst2, zeros64)
  return _tc3(parts2, rs_in, b2.reshape(1, D_OUT))


# speedup vs baseline: 1.0814x; 1.0814x over previous
import functools

import jax
import jax.numpy as jnp
from jax import lax
from jax.experimental import pallas as pl
from jax.experimental.pallas import tpu as pltpu
from jax.experimental.pallas import tpu_sc as plsc

N = 10000
E = 320000
D_IN = 128
D_H = 128
D_OUT = 64

NC = 2
NS = 16
NW = NC * NS

CHUNK = 128
R_PER_TILE = 80
EROWS = R_PER_TILE * NW
E_PAD = EROWS * CHUNK

ROWS_PER_TILE = 640
N_ACC = NS * ROWS_PER_TILE

_MESH = plsc.VectorSubcoreMesh(
    core_axis_name="c", subcore_axis_name="s", num_cores=NC, num_subcores=NS)


@functools.partial(
    pl.kernel,
    out_type=(jax.ShapeDtypeStruct((NC, N_ACC, 16), jnp.float32),
              jax.ShapeDtypeStruct((NC, N_ACC, 16), jnp.float32)),
    mesh=_MESH,
    compiler_params=pltpu.CompilerParams(use_tc_tiling_on_sc=False),
    scratch_types=[
        pltpu.VMEM((R_PER_TILE, CHUNK), jnp.int32),
        pltpu.VMEM((R_PER_TILE, CHUNK), jnp.int32),
        pltpu.VMEM((CHUNK, 16), jnp.float32),
        pltpu.VMEM_SHARED((N_ACC, 16), jnp.float32),
        pltpu.VMEM_SHARED((N_ACC, 16), jnp.float32),
        [pltpu.SemaphoreType.DMA] * 4,
        [pltpu.SemaphoreType.DMA] * 4,
    ],
)
def _deg_kernel(src2_hbm, dst2_hbm, ones_hbm, zeros_hbm, dout_hbm, din_hbm,
                sidx, didx, ones_v, acc_out, acc_in, osems, isems):
  cid = lax.axis_index("c")
  sid = lax.axis_index("s")
  wid = cid * NS + sid
  base = sid * ROWS_PER_TILE
  rbase = wid * R_PER_TILE

  def start_scatters(i, s):
    pltpu.async_copy(ones_v, acc_out.at[sidx.at[i]], osems[s], add=True)
    pltpu.async_copy(ones_v, acc_in.at[didx.at[i]], isems[s], add=True)

  def wait_scatters(s):
    pltpu.make_async_copy(ones_v, acc_out.at[sidx.at[0]], osems[s]).wait()
    pltpu.make_async_copy(ones_v, acc_in.at[didx.at[0]], isems[s]).wait()

  pltpu.sync_copy(src2_hbm.at[pl.ds(rbase, R_PER_TILE)], sidx)
  pltpu.sync_copy(dst2_hbm.at[pl.ds(rbase, R_PER_TILE)], didx)
  pltpu.sync_copy(ones_hbm, ones_v)
  pltpu.sync_copy(zeros_hbm, acc_out.at[pl.ds(base, ROWS_PER_TILE)])
  pltpu.sync_copy(zeros_hbm, acc_in.at[pl.ds(base, ROWS_PER_TILE)])
  plsc.subcore_barrier()

  def loop_body(g, _):
    for k in range(4):
      i = 4 * g + k

      @pl.when(i >= 4)
      def _():
        wait_scatters(k)

      start_scatters(i, k)
    return 0

  lax.fori_loop(0, R_PER_TILE // 4, loop_body, 0)

  for k in range(4):
    wait_scatters(k)

  plsc.subcore_barrier()
  pltpu.sync_copy(acc_out.at[pl.ds(base, ROWS_PER_TILE)],
                  dout_hbm.at[cid, pl.ds(base, ROWS_PER_TILE)])
  pltpu.sync_copy(acc_in.at[pl.ds(base, ROWS_PER_TILE)],
                  din_hbm.at[cid, pl.ds(base, ROWS_PER_TILE)])


def _make_scatter_kernel(D):
  params = None if D % 128 == 0 else pltpu.CompilerParams(
      use_tc_tiling_on_sc=False)

  @functools.partial(
      pl.kernel,
      out_type=jax.ShapeDtypeStruct((NC, N_ACC, D), jnp.float32),
      mesh=_MESH,
      compiler_params=params,
      scratch_types=[
          pltpu.VMEM((2, CHUNK), jnp.int32),
          pltpu.VMEM((R_PER_TILE, CHUNK), jnp.int32),
          pltpu.VMEM((CHUNK, D), jnp.float32),
          pltpu.VMEM((CHUNK, D), jnp.float32),
          pltpu.VMEM_SHARED((N_ACC, D), jnp.float32),
          [pltpu.SemaphoreType.DMA] * 2,
          [pltpu.SemaphoreType.DMA] * 2,
          [pltpu.SemaphoreType.DMA] * 2,
      ],
  )
  def _scatter_kernel(tab_hbm, src_hbm, dst2_hbm, zeros_hbm, out_hbm,
                      sidx, didx, rows0, rows1, acc, psems, gsems, ssems):
    cid = lax.axis_index("c")
    sid = lax.axis_index("s")
    wid = cid * NS + sid
    base = sid * ROWS_PER_TILE
    rbase = wid * R_PER_TILE
    ebase = rbase * CHUNK

    rows = (rows0, rows1)

    def prefetch_sidx(i, s):
      pltpu.async_copy(src_hbm.at[pl.ds(ebase + i * CHUNK, CHUNK)],
                       sidx.at[s], psems[s])

    def wait_sidx(s):
      pltpu.make_async_copy(src_hbm.at[pl.ds(ebase, CHUNK)], sidx.at[s],
                            psems[s]).wait()

    def start_gather(s):
      pltpu.async_copy(tab_hbm.at[sidx.at[s]], rows[s], gsems[s])

    def wait_gather(s):
      pltpu.make_async_copy(tab_hbm.at[sidx.at[s]], rows[s], gsems[s]).wait()

    def start_scatter(i, s):
      pltpu.async_copy(rows[s], acc.at[didx.at[i]], ssems[s], add=True)

    def wait_scatter(s):
      pltpu.make_async_copy(rows[s], acc.at[didx.at[0]], ssems[s]).wait()

    pltpu.sync_copy(src_hbm.at[pl.ds(ebase, CHUNK)], sidx.at[0])
    start_gather(0)
    pltpu.sync_copy(src_hbm.at[pl.ds(ebase + CHUNK, CHUNK)], sidx.at[1])
    pltpu.sync_copy(dst2_hbm.at[pl.ds(rbase, R_PER_TILE)], didx)
    pltpu.sync_copy(zeros_hbm, acc.at[pl.ds(base, ROWS_PER_TILE)])
    plsc.subcore_barrier()

    def loop_body(g, _):
      for k in range(2):
        i = 2 * g + k
        s = k
        o = 1 - k

        wait_gather(s)

        @pl.when(i + 2 < R_PER_TILE)
        def _():
          prefetch_sidx(i + 2, s)

        start_scatter(i, s)

        @pl.when(i + 1 < R_PER_TILE)
        def _():
          @pl.when(i >= 1)
          def _():
            wait_scatter(o)
            wait_sidx(o)

          start_gather(o)
      return 0

    lax.fori_loop(0, R_PER_TILE // 2, loop_body, 0)

    wait_scatter(0)
    wait_scatter(1)

    plsc.subcore_barrier()
    pltpu.sync_copy(acc.at[pl.ds(base, ROWS_PER_TILE)],
                    out_hbm.at[cid, pl.ds(base, ROWS_PER_TILE)])

  return _scatter_kernel


_scatter_128 = _make_scatter_kernel(D_H)
_scatter_64 = _make_scatter_kernel(D_OUT)


_BLK = 400
_GRID = N // _BLK


def _tc1_body(x_ref, w_ref, dout_ref, din_ref, xw_ref, rsin_ref, rsout_ref):
  rs_out = lax.rsqrt(jnp.maximum(dout_ref[0] + dout_ref[1], 1.0))
  rs_in = lax.rsqrt(jnp.maximum(din_ref[0] + din_ref[1], 1.0))
  rsout_ref[...] = rs_out
  rsin_ref[...] = rs_in
  xw = jnp.dot(x_ref[...], w_ref[...], preferred_element_type=jnp.float32)
  xw_ref[...] = xw * rs_out[:, 0:1]


def _tc1(x, w1, dout, din):
  return pl.pallas_call(
      _tc1_body,
      grid=(_GRID,),
      in_specs=[
          pl.BlockSpec((_BLK, D_IN), lambda i: (i, 0)),
          pl.BlockSpec((D_IN, D_H), lambda i: (0, 0)),
          pl.BlockSpec((NC, _BLK, 16), lambda i: (0, i, 0)),
          pl.BlockSpec((NC, _BLK, 16), lambda i: (0, i, 0)),
      ],
      out_specs=[
          pl.BlockSpec((_BLK, D_H), lambda i: (i, 0)),
          pl.BlockSpec((_BLK, 16), lambda i: (i, 0)),
          pl.BlockSpec((_BLK, 16), lambda i: (i, 0)),
      ],
      out_shape=[
          jax.ShapeDtypeStruct((N, D_H), jnp.float32),
          jax.ShapeDtypeStruct((N, 16), jnp.float32),
          jax.ShapeDtypeStruct((N, 16), jnp.float32),
      ],
  )(x, w1, dout, din)


def _tc2_body(p_ref, rsin_ref, b1_ref, w2_ref, rsout_ref, xw2_ref):
  agg = (p_ref[0] + p_ref[1]) * rsin_ref[...][:, 0:1]
  h = jnp.maximum(agg + b1_ref[...], 0.0)
  xw2 = jnp.dot(h, w2_ref[...], preferred_element_type=jnp.float32)
  xw2_ref[...] = xw2 * rsout_ref[...][:, 0:1]


def _tc2(parts1, rs_in, b1, w2, rs_out):
  return pl.pallas_call(
      _tc2_body,
      grid=(_GRID,),
      in_specs=[
          pl.BlockSpec((NC, _BLK, D_H), lambda i: (0, i, 0)),
          pl.BlockSpec((_BLK, 16), lambda i: (i, 0)),
          pl.BlockSpec((1, D_H), lambda i: (0, 0)),
          pl.BlockSpec((D_H, D_OUT), lambda i: (0, 0)),
          pl.BlockSpec((_BLK, 16), lambda i: (i, 0)),
      ],
      out_specs=pl.BlockSpec((_BLK, D_OUT), lambda i: (i, 0)),
      out_shape=jax.ShapeDtypeStruct((N, D_OUT), jnp.float32),
  )(parts1, rs_in, b1, w2, rs_out)


def _tc3_body(p_ref, rsin_ref, b2_ref, out_ref):
  agg = (p_ref[0] + p_ref[1]) * rsin_ref[...][:, 0:1]
  out_ref[...] = agg + b2_ref[...]


def _tc3(parts2, rs_in, b2):
  return pl.pallas_call(
      _tc3_body,
      grid=(_GRID,),
      in_specs=[
          pl.BlockSpec((NC, _BLK, D_OUT), lambda i: (0, i, 0)),
          pl.BlockSpec((_BLK, 16), lambda i: (i, 0)),
          pl.BlockSpec((1, D_OUT), lambda i: (0, 0)),
      ],
      out_specs=pl.BlockSpec((_BLK, D_OUT), lambda i: (i, 0)),
      out_shape=jax.ShapeDtypeStruct((N, D_OUT), jnp.float32),
  )(parts2, rs_in, b2)


def kernel(inputs, edge_index, W1, b1, W2, b2):
  pad = E_PAD - E
  trash = N + (jnp.arange(pad, dtype=jnp.int32) % 128)
  src = jnp.concatenate([edge_index[0],
                         jnp.arange(pad, dtype=jnp.int32) % 128])
  dst2 = jnp.concatenate([edge_index[1], trash]).reshape(EROWS, CHUNK)
  src2_deg = jnp.concatenate([edge_index[0], trash]).reshape(EROWS, CHUNK)
  ones16 = jnp.ones((CHUNK, 16), jnp.float32)
  zeros16 = jnp.zeros((ROWS_PER_TILE, 16), jnp.float32)
  zeros128 = jnp.zeros((ROWS_PER_TILE, D_H), jnp.float32)
  zeros64 = jnp.zeros((ROWS_PER_TILE, D_OUT), jnp.float32)

  dout, din = _deg_kernel(src2_deg, dst2, ones16, zeros16)
  xw1s, rs_in, rs_out = _tc1(inputs, W1, dout, din)
  parts1 = _scatter_128(xw1s, src, dst2, zeros128)
  xw2s = _tc2(parts1, rs_in, b1.reshape(1, D_H), W2, rs_out)
  parts2 = _scatter_64(xw2s, src, dst2, zeros64)
  return _tc3(parts2, rs_in, b2.reshape(1, D_OUT))
